# baseline (speedup 1.0000x reference)
import functools

import jax
import jax.numpy as jnp
from jax import lax
from jax.experimental import pallas as pl
from jax.experimental.pallas import tpu as pltpu
from jax.experimental.pallas import tpu_sc as plsc

_PAD = 8
_D = 300
_DP = 320
_NV = 9
_NP = _NV * _NV
_RSTR = 8
_ROW = 2 * _DP
_NW = 32
_CHUNK = 64


def _sc_pair_gather(tbl_flat, pidx, n_pair):
  per_w = n_pair // _NW
  n_chunk = per_w // _CHUNK
  mesh = plsc.VectorSubcoreMesh(core_axis_name="c", subcore_axis_name="s")

  @functools.partial(
      pl.kernel,
      out_type=jax.ShapeDtypeStruct((n_pair, _ROW), jnp.float32),
      mesh=mesh,
      scratch_types=[
          pltpu.VMEM((_NV * _DP,), jnp.float32),
          pltpu.VMEM_SHARED((_NP * _RSTR, _ROW), jnp.float32),
          pltpu.VMEM((n_chunk, _CHUNK), jnp.int32),
          pltpu.VMEM((_CHUNK, _ROW), jnp.float32),
          pltpu.VMEM((_CHUNK, _ROW), jnp.float32),
          pltpu.SemaphoreType.DMA,
          pltpu.SemaphoreType.DMA,
          pltpu.SemaphoreType.DMA,
          pltpu.SemaphoreType.DMA,
      ],
      compiler_params=pltpu.CompilerParams(use_tc_tiling_on_sc=False),
  )
  def k(tbl_hbm, idx_hbm, out_hbm, tbl_v, pt_sh, idx_v, buf0, buf1,
        g0, g1, w0, w1):
    cid = lax.axis_index("c")
    sid = lax.axis_index("s")
    wid = sid * 2 + cid
    base = wid * per_w

    pltpu.sync_copy(tbl_hbm, tbl_v)
    pltpu.sync_copy(idx_hbm.at[wid], idx_v)

    def fillp(t, carry):
      p = t * 16 + sid

      @pl.when(p < _NP)
      def _():
        d0 = p // _NV
        d1 = p % _NV
        s0_off = pl.multiple_of(d0 * _DP, 8)
        s1_off = pl.multiple_of(d1 * _DP, 8)
        row = pl.multiple_of(p * _RSTR, 8)
        pltpu.sync_copy(
            tbl_v.at[pl.ds(s0_off, _DP)], pt_sh.at[row, pl.ds(0, _DP)]
        )
        pltpu.sync_copy(
            tbl_v.at[pl.ds(s1_off, _DP)], pt_sh.at[row, pl.ds(_DP, _DP)]
        )

      return carry

    lax.fori_loop(0, (_NP + 15) // 16, fillp, 0)
    plsc.subcore_barrier()

    def gather(j, buf, sem):
      return pltpu.async_copy(pt_sh.at[idx_v.at[j]], buf, sem)

    def write(j, buf, sem):
      return pltpu.async_copy(
          buf, out_hbm.at[pl.ds(base + j * _CHUNK, _CHUNK)], sem
      )

    def body(t, carry):
      hg0 = gather(2 * t, buf0, g0)
      hg0.wait()
      hw0 = write(2 * t, buf0, w0)
      hg1 = gather(2 * t + 1, buf1, g1)
      hg1.wait()
      hw1 = write(2 * t + 1, buf1, w1)
      hw0.wait()
      hw1.wait()
      return carry

    lax.fori_loop(0, n_chunk // 2, body, 0)

  return k(tbl_flat, pidx)


def kernel(inputs, emb_table):
  b, l = inputs.shape
  n_tok = b * l
  n_pair = n_tok // 2
  per_w = n_pair // _NW

  tbl = emb_table.at[0].set(0.0).at[_PAD].set(0.0)
  tbl_flat = jnp.pad(tbl, ((0, 0), (0, _DP - _D))).reshape(-1)

  ip = inputs.reshape(-1, 2).astype(jnp.int32)
  pidx = (ip[:, 0] * _NV + ip[:, 1]) * _RSTR
  pidx = pidx.reshape(_NW, per_w // _CHUNK, _CHUNK)

  out = _sc_pair_gather(tbl_flat, pidx, n_pair)
  out = out.reshape(n_pair, 2, _DP)[:, :, :_D]
  return out.reshape(b, l, _D)


# speedup vs baseline: 1.9353x; 1.9353x over previous
import functools

import jax
import jax.numpy as jnp
from jax import lax
from jax.experimental import pallas as pl
from jax.experimental.pallas import tpu as pltpu
from jax.experimental.pallas import tpu_sc as plsc

_PAD = 8
_D = 300
_P = 2
_PD = _D * _P
_PDP = 608
_NV = 9
_NW = 32
_CHUNK = 64


def _sc_gather(ptbl, pidx, n_pair):
  per_w = n_pair // _NW
  n_chunk = per_w // _CHUNK
  mesh = plsc.VectorSubcoreMesh(core_axis_name="c", subcore_axis_name="s")

  @functools.partial(
      pl.kernel,
      out_type=jax.ShapeDtypeStruct((n_pair, _PD), jnp.float32),
      mesh=mesh,
      scratch_types=[
          pltpu.VMEM((n_chunk, _CHUNK), jnp.int32),
          pltpu.VMEM((_CHUNK, _PDP), jnp.float32),
          pltpu.VMEM((_CHUNK, _PDP), jnp.float32),
          pltpu.SemaphoreType.DMA,
          pltpu.SemaphoreType.DMA,
          pltpu.SemaphoreType.DMA,
          pltpu.SemaphoreType.DMA,
      ],
      compiler_params=pltpu.CompilerParams(use_tc_tiling_on_sc=False),
  )
  def k(tbl_hbm, idx_hbm, out_hbm, idx_v, buf0, buf1, g0, g1, w0, w1):
    wid = lax.axis_index("s") * 2 + lax.axis_index("c")
    base = wid * per_w
    pltpu.sync_copy(idx_hbm.at[wid], idx_v)

    def gather(j, buf, sem):
      return pltpu.async_copy(tbl_hbm.at[idx_v.at[j]], buf, sem)

    def write(j, buf, sem):
      return pltpu.async_copy(
          buf.at[:, pl.ds(0, _PD)],
          out_hbm.at[pl.ds(base + j * _CHUNK, _CHUNK)],
          sem,
      )

    def body(t, carry):
      hg0 = gather(2 * t, buf0, g0)
      hg0.wait()
      hw0 = write(2 * t, buf0, w0)
      hg1 = gather(2 * t + 1, buf1, g1)
      hg1.wait()
      hw1 = write(2 * t + 1, buf1, w1)
      hw0.wait()
      hw1.wait()
      return carry

    lax.fori_loop(0, n_chunk // 2, body, 0)

  return k(ptbl, pidx)


def kernel(inputs, emb_table):
  b, l = inputs.shape
  n_tok = b * l
  n_pair = n_tok // _P
  per_w = n_pair // _NW
  n = _NV

  tbl = emb_table.at[0].set(0.0).at[_PAD].set(0.0)
  span = jnp.arange(n * n, dtype=jnp.int32)
  pt = jnp.concatenate(
      [jnp.take(tbl, span // n, axis=0), jnp.take(tbl, span % n, axis=0)],
      axis=1,
  )
  pt = jnp.pad(pt, ((0, 0), (0, _PDP - _PD)))

  ip = inputs.reshape(-1, _P).astype(jnp.int32)
  pidx = ip[:, 0] * n + ip[:, 1]
  pidx = pidx.reshape(_NW, per_w // _CHUNK, _CHUNK)

  out = _sc_gather(pt, pidx, n_pair)
  return out.reshape(b, l, _D)


# speedup vs baseline: 2.0275x; 1.0476x over previous
import functools

import jax
import jax.numpy as jnp
from jax import lax
from jax.experimental import pallas as pl
from jax.experimental.pallas import tpu as pltpu
from jax.experimental.pallas import tpu_sc as plsc

_PAD = 8
_D = 300
_Q = 4
_QD = _D * _Q
_NV = 9
_NW = 32
_CHUNK = 32


def _sc_gather(qtbl, qidx, n_quad):
  per_w = n_quad // _NW
  n_chunk = per_w // _CHUNK
  mesh = plsc.VectorSubcoreMesh(core_axis_name="c", subcore_axis_name="s")

  @functools.partial(
      pl.kernel,
      out_type=jax.ShapeDtypeStruct((n_quad, _QD), jnp.float32),
      mesh=mesh,
      scratch_types=[
          pltpu.VMEM((n_chunk, _CHUNK), jnp.int32),
          pltpu.VMEM((_CHUNK, _QD), jnp.float32),
          pltpu.VMEM((_CHUNK, _QD), jnp.float32),
          pltpu.SemaphoreType.DMA,
          pltpu.SemaphoreType.DMA,
          pltpu.SemaphoreType.DMA,
          pltpu.SemaphoreType.DMA,
      ],
      compiler_params=pltpu.CompilerParams(use_tc_tiling_on_sc=False),
  )
  def k(tbl_hbm, idx_hbm, out_hbm, idx_v, buf0, buf1, g0, g1, w0, w1):
    wid = lax.axis_index("s") * 2 + lax.axis_index("c")
    base = wid * per_w
    pltpu.sync_copy(idx_hbm.at[wid], idx_v)

    def gather(j, buf, sem):
      return pltpu.async_copy(tbl_hbm.at[idx_v.at[j]], buf, sem)

    def write(j, buf, sem):
      return pltpu.async_copy(
          buf, out_hbm.at[pl.ds(base + j * _CHUNK, _CHUNK)], sem
      )

    def body(t, carry):
      hg0 = gather(2 * t, buf0, g0)
      hg0.wait()
      hw0 = write(2 * t, buf0, w0)
      hg1 = gather(2 * t + 1, buf1, g1)
      hg1.wait()
      hw1 = write(2 * t + 1, buf1, w1)
      hw0.wait()
      hw1.wait()
      return carry

    lax.fori_loop(0, n_chunk // 2, body, 0)

  return k(qtbl, qidx)


def kernel(inputs, emb_table):
  b, l = inputs.shape
  n_tok = b * l
  n_quad = n_tok // _Q
  per_w = n_quad // _NW

  tbl = emb_table.at[0].set(0.0).at[_PAD].set(0.0)
  n = _NV
  span = jnp.arange(n * n * n * n, dtype=jnp.int32)
  qt = jnp.concatenate(
      [
          jnp.take(tbl, (span // (n * n * n)) % n, axis=0),
          jnp.take(tbl, (span // (n * n)) % n, axis=0),
          jnp.take(tbl, (span // n) % n, axis=0),
          jnp.take(tbl, span % n, axis=0),
      ],
      axis=1,
  )

  iq = inputs.reshape(-1, _Q).astype(jnp.int32)
  qidx = ((iq[:, 0] * n + iq[:, 1]) * n + iq[:, 2]) * n + iq[:, 3]
  qidx = qidx.reshape(_NW, per_w // _CHUNK, _CHUNK)

  out = _sc_gather(qt, qidx, n_quad)
  return out.reshape(b, l, _D)
